# Initial kernel scaffold; baseline (speedup 1.0000x reference)
import jax
import jax.numpy as jnp
from jax.experimental import pallas as pl


def kernel(inputs):
    raise NotImplementedError("write your pallas kernel here")


# speedup vs baseline: 14.2795x; 14.2795x over previous
import jax
import jax.numpy as jnp
from jax.experimental import pallas as pl

K = 16
N = 8192
D = 64
ROWS = 256


def _knn_block(xb_ref, xf_ref, sqb_ref, sqf_ref, out_ref):
    xb = xb_ref[:]
    xf = xf_ref[:]
    sqb = sqb_ref[:]
    sqf = sqf_ref[:]
    mm = jax.lax.dot_general(
        xb, xf,
        dimension_numbers=(((1,), (1,)), ((), ())),
        preferred_element_type=jnp.float32,
    )
    d2 = (sqb + sqf) - 2.0 * mm
    d = jnp.sqrt(jnp.maximum(d2, 1e-12))

    iota = jax.lax.broadcasted_iota(jnp.int32, (ROWS, N), 1)
    big = jnp.int32(N)
    for k in range(K + 1):
        m = jnp.min(d, axis=1, keepdims=True)
        idx = jnp.min(jnp.where(d == m, iota, big), axis=1,
                      keepdims=True)
        if k > 0:
            out_ref[:, k - 1:k] = idx
        d = jnp.where(iota == idx, jnp.inf, d)


def kernel(inputs):
    x = inputs
    sq = jnp.sum(x * x, axis=1)
    xbf = x.astype(jnp.bfloat16)
    grid = (N // ROWS,)
    return pl.pallas_call(
        _knn_block,
        grid=grid,
        in_specs=[
            pl.BlockSpec((ROWS, D), lambda i: (i, 0)),
            pl.BlockSpec((N, D), lambda i: (0, 0)),
            pl.BlockSpec((ROWS, 1), lambda i: (i, 0)),
            pl.BlockSpec((1, N), lambda i: (0, 0)),
        ],
        out_specs=pl.BlockSpec((ROWS, K), lambda i: (i, 0)),
        out_shape=jax.ShapeDtypeStruct((N, K), jnp.int32),
    )(xbf, xbf, sq[:, None], sq[None, :])


# speedup vs baseline: 19.0219x; 1.3321x over previous
import jax
import jax.numpy as jnp
from jax.experimental import pallas as pl

K = 16
N = 8192
D = 64
ROWS = 256


def _knn_block(xb_ref, xf_ref, sqb_ref, sqf_ref, out_ref):
    xb = xb_ref[:]
    xf = xf_ref[:]
    sqb = sqb_ref[:]
    sqf = sqf_ref[:]
    mm = jax.lax.dot_general(
        xb, xf,
        dimension_numbers=(((1,), (1,)), ((), ())),
        preferred_element_type=jnp.float32,
    )
    d2 = (sqb + sqf) - 2.0 * mm
    d = jnp.sqrt(jnp.maximum(d2, 1e-12))

    iota = jax.lax.broadcasted_iota(jnp.int32, (ROWS, N), 1)
    for k in range(K + 1):
        idx = jnp.argmin(d, axis=1).astype(jnp.int32)[:, None]
        if k > 0:
            out_ref[:, k - 1:k] = idx
        d = jnp.where(iota == idx, jnp.inf, d)


def kernel(inputs):
    x = inputs
    sq = jnp.sum(x * x, axis=1)
    xbf = x.astype(jnp.bfloat16)
    grid = (N // ROWS,)
    return pl.pallas_call(
        _knn_block,
        grid=grid,
        in_specs=[
            pl.BlockSpec((ROWS, D), lambda i: (i, 0)),
            pl.BlockSpec((N, D), lambda i: (0, 0)),
            pl.BlockSpec((ROWS, 1), lambda i: (i, 0)),
            pl.BlockSpec((1, N), lambda i: (0, 0)),
        ],
        out_specs=pl.BlockSpec((ROWS, K), lambda i: (i, 0)),
        out_shape=jax.ShapeDtypeStruct((N, K), jnp.int32),
    )(xbf, xbf, sq[:, None], sq[None, :])
